# baseline (speedup 1.0000x reference)
import functools

import jax
import jax.numpy as jnp
import numpy as np
from jax import lax
from jax.experimental import pallas as pl
from jax.experimental.pallas import tpu as pltpu
from jax.experimental.pallas import tpu_sc as plsc

CHUNK = 80
NPB_C = 1000


def _sc_segment_sum(src3, edge_attr, zeros_rows, zeros_n, n_nodes):
    NW, chunks, chunk = src3.shape
    E, D = edge_attr.shape
    info = plsc.get_sparse_core_info()
    NC, NS = info.num_cores, info.num_subcores
    assert NW == NC * NS
    EPW = E // NW
    NCH = n_nodes // chunk
    TREP = (NCH + NS - 1) // NS
    NV = chunk // 16

    mesh = plsc.VectorSubcoreMesh(core_axis_name="c", subcore_axis_name="s")

    @functools.partial(
        pl.kernel,
        out_type=(
            pltpu.HBM((NC, n_nodes, D), jnp.float32),
            pltpu.HBM((NW, n_nodes), jnp.float32),
        ),
        mesh=mesh,
        scratch_types=[
            pltpu.VMEM((chunks, chunk), jnp.int32),
            pltpu.VMEM((chunk, D), jnp.float32),
            pltpu.VMEM((chunk, D), jnp.float32),
            pltpu.VMEM((n_nodes,), jnp.float32),
            pltpu.VMEM_SHARED((n_nodes, D), jnp.float32),
            pltpu.SemaphoreType.DMA,
            pltpu.SemaphoreType.DMA,
        ],
        compiler_params=pltpu.CompilerParams(needs_layout_passes=False),
    )
    def k(idx_hbm, ea_hbm, zrows_hbm, zn_hbm, acc_out, cnt_out,
          idx_v, rows0_v, rows1_v, cnt_v, acc_sh, sem0, sem1):
        cid = lax.axis_index("c")
        sid = lax.axis_index("s")
        wid = sid * NC + cid

        pltpu.sync_copy(idx_hbm.at[wid], idx_v)
        pltpu.sync_copy(zn_hbm, cnt_v)
        for t in range(TREP):
            c = sid + t * NS
            if (t + 1) * NS <= NCH:
                pltpu.async_copy(zrows_hbm, acc_sh.at[pl.ds(c * chunk, chunk)],
                                 sem0)
            else:
                @pl.when(c < NCH)
                def _():
                    pltpu.async_copy(
                        zrows_hbm, acc_sh.at[pl.ds(c * chunk, chunk)], sem0)
        for t in range(TREP):
            c = sid + t * NS
            if (t + 1) * NS <= NCH:
                pltpu.make_async_copy(
                    zrows_hbm, acc_sh.at[pl.ds(c * chunk, chunk)], sem0).wait()
            else:
                @pl.when(c < NCH)
                def _():
                    pltpu.make_async_copy(
                        zrows_hbm, acc_sh.at[pl.ds(c * chunk, chunk)],
                        sem0).wait()
        plsc.subcore_barrier()

        base = wid * EPW
        ones16 = jnp.full((16,), 1.0, jnp.float32)

        def _ea(j):
            return ea_hbm.at[pl.ds(base + j * chunk, chunk)]

        def _counts(j):
            for l in range(NV):
                vec = idx_v[j, pl.ds(l * 16, 16)]
                plsc.addupdate_scatter(cnt_v, [vec], ones16)

        pltpu.async_copy(_ea(0), rows0_v, sem0)
        npairs = chunks // 2

        def body(t, carry):
            j0 = 2 * t
            pltpu.make_async_copy(_ea(j0), rows0_v, sem0).wait()
            pltpu.async_copy(_ea(j0 + 1), rows1_v, sem1)
            pltpu.sync_copy(rows0_v, acc_sh.at[idx_v.at[j0]], add=True)
            _counts(j0)
            pltpu.make_async_copy(_ea(j0 + 1), rows1_v, sem1).wait()
            @pl.when(j0 + 2 < chunks)
            def _():
                pltpu.async_copy(_ea(j0 + 2), rows0_v, sem0)
            pltpu.sync_copy(rows1_v, acc_sh.at[idx_v.at[j0 + 1]], add=True)
            _counts(j0 + 1)
            return carry

        lax.fori_loop(0, npairs, body, 0)
        if chunks % 2:
            j = chunks - 1
            pltpu.make_async_copy(_ea(j), rows0_v, sem0).wait()
            pltpu.sync_copy(rows0_v, acc_sh.at[idx_v.at[j]], add=True)
            _counts(j)
        plsc.subcore_barrier()

        def _oc(c):
            return (acc_sh.at[pl.ds(c * chunk, chunk)],
                    acc_out.at[cid, pl.ds(c * chunk, chunk)])

        pltpu.async_copy(cnt_v, cnt_out.at[wid], sem1)
        for t in range(TREP):
            c = sid + t * NS
            if (t + 1) * NS <= NCH:
                pltpu.async_copy(*_oc(c), sem0)
            else:
                @pl.when(c < NCH)
                def _():
                    pltpu.async_copy(*_oc(c), sem0)
        for t in range(TREP):
            c = sid + t * NS
            if (t + 1) * NS <= NCH:
                pltpu.make_async_copy(*_oc(c), sem0).wait()
            else:
                @pl.when(c < NCH)
                def _():
                    pltpu.make_async_copy(*_oc(c), sem0).wait()
        pltpu.make_async_copy(cnt_v, cnt_out.at[wid], sem1).wait()

    return k(src3, edge_attr, zeros_rows, zeros_n)


def _leaky(v):
    return jnp.where(v >= 0, v, 0.01 * v)


def _dense_body(*refs):
    (xr, p0r, p1r, cpr, idxr) = refs[:5]
    out_ref = refs[-1]
    w = [r[...] for r in refs[5:-1]]
    (w0, b0, w0c, b0c, g0, be0,
     w1, b1, w1c, b1c, g1, be1,
     w2, b2, w2c, b2c, g2, be2,
     tW1, tb1, bng, bnb, tW2, tb2,
     sW1, sb1, s1g, s1b, sW2, sb2, s2g, s2b, sW3, sb3) = w
    d = xr.shape[1]
    layers = ((w0[:d], w0[d:], b0, w0c, b0c, g0, be0),
              (w1[:d], w1[d:], b1, w1c, b1c, g1, be1),
              (w2[:d], w2[d:], b2, w2c, b2c, g2, be2))

    n = xr.shape[0]
    parts = cpr[...]
    cnt_col = lax.dot_general(parts, jnp.ones((parts.shape[0], 1), jnp.float32),
                              (((0,), (0,)), ((), ())),
                              preferred_element_type=jnp.float32)
    cnt = jnp.maximum(cnt_col, 1.0)
    ea_mean = (p0r[...] + p1r[...]) / cnt

    h = xr[...]
    for (Wa, Wb, bb, Wc, bc, g, be) in layers:
        agg = h * ea_mean
        z = _leaky(jnp.dot(h, Wa, preferred_element_type=jnp.float32)
                   + jnp.dot(agg, Wb, preferred_element_type=jnp.float32) + bb)
        z = jnp.dot(z, Wc, preferred_element_type=jnp.float32) + bc
        mu = jnp.mean(z, axis=-1, keepdims=True)
        var = jnp.mean((z - mu) * (z - mu), axis=-1, keepdims=True)
        h = (z - mu) * lax.rsqrt(var + 1e-5) * g + be

    v = jnp.dot(h, tW1, preferred_element_type=jnp.float32) + tb1
    mu0 = jnp.mean(v, axis=0, keepdims=True)
    var0 = jnp.mean((v - mu0) * (v - mu0), axis=0, keepdims=True)
    v = _leaky((v - mu0) * lax.rsqrt(var0 + 1e-5) * bng + bnb)
    v = jnp.dot(v, tW2, preferred_element_type=jnp.float32) + tb2

    nb = n // NPB_C
    idx = idxr[...]
    ii = lax.broadcasted_iota(jnp.int32, (n, idx.shape[1]), 0)
    csel = jnp.zeros(ii.shape, jnp.bool_)
    for b in range(nb):
        csel = jnp.logical_or(csel, ii == (idx + b * NPB_C))
    M = v * csel.astype(jnp.float32)
    ib = lax.broadcasted_iota(jnp.int32, (nb, n), 0)
    ir = lax.broadcasted_iota(jnp.int32, (nb, n), 1)
    diff = ir - ib * NPB_C
    bsel = jnp.logical_and(diff >= 0, diff < NPB_C).astype(jnp.float32)
    sn = jnp.dot(bsel, M, preferred_element_type=jnp.float32)

    u = jnp.dot(sn, sW1, preferred_element_type=jnp.float32) + sb1
    mu1 = jnp.mean(u, axis=0, keepdims=True)
    var1 = jnp.mean((u - mu1) * (u - mu1), axis=0, keepdims=True)
    u = _leaky((u - mu1) * lax.rsqrt(var1 + 1e-5) * s1g + s1b)
    u = jnp.dot(u, sW2, preferred_element_type=jnp.float32) + sb2
    mu2 = jnp.mean(u, axis=0, keepdims=True)
    var2 = jnp.mean((u - mu2) * (u - mu2), axis=0, keepdims=True)
    u = _leaky((u - mu2) * lax.rsqrt(var2 + 1e-5) * s2g + s2b)
    out_ref[...] = jnp.dot(u, sW3, preferred_element_type=jnp.float32) + sb3


def _dense_call(x, p0, p1, cnt_parts, sn_idx, wlist):
    nb = x.shape[0] // NPB_C
    return pl.pallas_call(
        _dense_body,
        out_shape=jax.ShapeDtypeStruct((nb, 2), jnp.float32),
    )(x, p0, p1, cnt_parts, sn_idx, *wlist)


def kernel(x, edge_index, edge_attr, batch, supernode_indices, params):
    n, d = x.shape
    e = edge_attr.shape[0]
    info = plsc.get_sparse_core_info()
    nw = info.num_cores * info.num_subcores
    chunks = e // (nw * CHUNK)
    assert chunks * nw * CHUNK == e

    src3 = edge_index[0].reshape(nw, chunks, CHUNK)
    zeros_rows = np.zeros((CHUNK, d), np.float32)
    zeros_n = np.zeros((n,), np.float32)
    acc, cnt_parts = _sc_segment_sum(src3, edge_attr, zeros_rows, zeros_n, n)

    wlist = []
    for p in params["proc"]:
        wlist += [p["W1"], p["b1"].reshape(1, -1),
                  p["W2"], p["b2"].reshape(1, -1),
                  p["ln_g"].reshape(1, -1), p["ln_b"].reshape(1, -1)]
    t = params["time"]
    wlist += [t["W1"], t["b1"].reshape(1, -1), t["bn_g"].reshape(1, -1),
              t["bn_b"].reshape(1, -1), t["W2"], t["b2"].reshape(1, -1)]
    s = params["super"]
    wlist += [s["W1"], s["b1"].reshape(1, -1), s["bn1_g"].reshape(1, -1),
              s["bn1_b"].reshape(1, -1), s["W2"], s["b2"].reshape(1, -1),
              s["bn2_g"].reshape(1, -1), s["bn2_b"].reshape(1, -1),
              s["W3"], s["b3"].reshape(1, -1)]

    sn_idx = supernode_indices.reshape(1, -1).astype(jnp.int32)
    return _dense_call(x, acc[0], acc[1], cnt_parts, sn_idx, wlist)


# speedup vs baseline: 1.0651x; 1.0651x over previous
import functools

import jax
import jax.numpy as jnp
import numpy as np
from jax import lax
from jax.experimental import pallas as pl
from jax.experimental.pallas import tpu as pltpu
from jax.experimental.pallas import tpu_sc as plsc

CHUNK = 80
NPB_C = 1000


def _sc_segment_sum(src3, edge_attr, zeros_rows, zeros_n, n_nodes):
    NW, chunks, chunk = src3.shape
    E, D = edge_attr.shape
    info = plsc.get_sparse_core_info()
    NC, NS = info.num_cores, info.num_subcores
    assert NW == NC * NS
    EPW = E // NW
    NCH = n_nodes // chunk
    TREP = (NCH + NS - 1) // NS
    NV = chunk // 16

    mesh = plsc.VectorSubcoreMesh(core_axis_name="c", subcore_axis_name="s")

    @functools.partial(
        pl.kernel,
        out_type=(
            pltpu.HBM((NC, n_nodes, D), jnp.float32),
            pltpu.HBM((NW, n_nodes), jnp.float32),
        ),
        mesh=mesh,
        scratch_types=[
            pltpu.VMEM((chunks, chunk), jnp.int32),
            pltpu.VMEM((chunk, D), jnp.float32),
            pltpu.VMEM((chunk, D), jnp.float32),
            pltpu.VMEM((n_nodes,), jnp.float32),
            pltpu.VMEM_SHARED((n_nodes, D), jnp.float32),
            pltpu.SemaphoreType.DMA,
            pltpu.SemaphoreType.DMA,
        ],
        compiler_params=pltpu.CompilerParams(needs_layout_passes=False),
    )
    def k(idx_hbm, ea_hbm, zrows_hbm, zn_hbm, acc_out, cnt_out,
          idx_v, rows0_v, rows1_v, cnt_v, acc_sh, sem0, sem1):
        cid = lax.axis_index("c")
        sid = lax.axis_index("s")
        wid = sid * NC + cid

        pltpu.sync_copy(idx_hbm.at[wid], idx_v)
        pltpu.sync_copy(zn_hbm, cnt_v)
        pltpu.sync_copy(zrows_hbm, rows0_v)

        def _zc(c):
            return (rows0_v, acc_sh.at[pl.ds(c * chunk, chunk)])

        for t in range(TREP):
            c = sid + t * NS
            if (t + 1) * NS <= NCH:
                pltpu.async_copy(*_zc(c), sem0)
            else:
                @pl.when(c < NCH)
                def _():
                    pltpu.async_copy(*_zc(c), sem0)
        for t in range(TREP):
            c = sid + t * NS
            if (t + 1) * NS <= NCH:
                pltpu.make_async_copy(*_zc(c), sem0).wait()
            else:
                @pl.when(c < NCH)
                def _():
                    pltpu.make_async_copy(*_zc(c), sem0).wait()
        plsc.subcore_barrier()

        base = wid * EPW
        ones16 = jnp.full((16,), 1.0, jnp.float32)

        def _ea(j):
            return ea_hbm.at[pl.ds(base + j * chunk, chunk)]

        def _counts(j):
            for l in range(NV):
                vec = idx_v[j, pl.ds(l * 16, 16)]
                plsc.addupdate_scatter(cnt_v, [vec], ones16)

        pltpu.async_copy(_ea(0), rows0_v, sem0)
        npairs = chunks // 2

        def body(t, carry):
            j0 = 2 * t
            pltpu.make_async_copy(_ea(j0), rows0_v, sem0).wait()
            pltpu.async_copy(_ea(j0 + 1), rows1_v, sem1)
            pltpu.sync_copy(rows0_v, acc_sh.at[idx_v.at[j0]], add=True)
            _counts(j0)
            pltpu.make_async_copy(_ea(j0 + 1), rows1_v, sem1).wait()
            @pl.when(j0 + 2 < chunks)
            def _():
                pltpu.async_copy(_ea(j0 + 2), rows0_v, sem0)
            pltpu.sync_copy(rows1_v, acc_sh.at[idx_v.at[j0 + 1]], add=True)
            _counts(j0 + 1)
            return carry

        lax.fori_loop(0, npairs, body, 0)
        if chunks % 2:
            j = chunks - 1
            pltpu.make_async_copy(_ea(j), rows0_v, sem0).wait()
            pltpu.sync_copy(rows0_v, acc_sh.at[idx_v.at[j]], add=True)
            _counts(j)
        plsc.subcore_barrier()

        bufs = (rows0_v, rows1_v)
        sems = (sem0, sem1)

        def _wr(c, b, sm):
            return (b, acc_out.at[cid, pl.ds(c * chunk, chunk)], sm)

        for t in range(TREP):
            c = sid + t * NS
            b, sm = bufs[t % 2], sems[t % 2]

            def _step(c=c, b=b, sm=sm, t=t):
                if t >= 2:
                    pltpu.make_async_copy(*_wr(c, b, sm)).wait()
                pltpu.sync_copy(acc_sh.at[pl.ds(c * chunk, chunk)], b)
                pltpu.async_copy(*_wr(c, b, sm))

            if (t + 1) * NS <= NCH:
                _step()
            else:
                pl.when(c < NCH)(_step)
        for t in range(TREP - 2, TREP):
            c = sid + t * NS
            b, sm = bufs[t % 2], sems[t % 2]
            if (t + 1) * NS <= NCH:
                pltpu.make_async_copy(*_wr(c, b, sm)).wait()
            else:
                @pl.when(c < NCH)
                def _():
                    pltpu.make_async_copy(*_wr(c, b, sm)).wait()
        pltpu.sync_copy(cnt_v, cnt_out.at[wid])

    return k(src3, edge_attr, zeros_rows, zeros_n)


def _leaky(v):
    return jnp.where(v >= 0, v, 0.01 * v)


def _dense_body(*refs):
    (xr, p0r, p1r, cpr, idxr) = refs[:5]
    out_ref = refs[-1]
    w = [r[...] for r in refs[5:-1]]
    (w0, b0, w0c, b0c, g0, be0,
     w1, b1, w1c, b1c, g1, be1,
     w2, b2, w2c, b2c, g2, be2,
     tW1, tb1, bng, bnb, tW2, tb2,
     sW1, sb1, s1g, s1b, sW2, sb2, s2g, s2b, sW3, sb3) = w
    d = xr.shape[1]
    layers = ((w0[:d], w0[d:], b0, w0c, b0c, g0, be0),
              (w1[:d], w1[d:], b1, w1c, b1c, g1, be1),
              (w2[:d], w2[d:], b2, w2c, b2c, g2, be2))

    n = xr.shape[0]
    parts = cpr[...]
    cnt_col = lax.dot_general(parts, jnp.ones((parts.shape[0], 1), jnp.float32),
                              (((0,), (0,)), ((), ())),
                              preferred_element_type=jnp.float32)
    cnt = jnp.maximum(cnt_col, 1.0)
    ea_mean = (p0r[...] + p1r[...]) / cnt

    h = xr[...]
    for (Wa, Wb, bb, Wc, bc, g, be) in layers:
        agg = h * ea_mean
        z = _leaky(jnp.dot(h, Wa, preferred_element_type=jnp.float32)
                   + jnp.dot(agg, Wb, preferred_element_type=jnp.float32) + bb)
        z = jnp.dot(z, Wc, preferred_element_type=jnp.float32) + bc
        mu = jnp.mean(z, axis=-1, keepdims=True)
        var = jnp.mean((z - mu) * (z - mu), axis=-1, keepdims=True)
        h = (z - mu) * lax.rsqrt(var + 1e-5) * g + be

    v = jnp.dot(h, tW1, preferred_element_type=jnp.float32) + tb1
    mu0 = jnp.mean(v, axis=0, keepdims=True)
    var0 = jnp.mean((v - mu0) * (v - mu0), axis=0, keepdims=True)
    v = _leaky((v - mu0) * lax.rsqrt(var0 + 1e-5) * bng + bnb)
    v = jnp.dot(v, tW2, preferred_element_type=jnp.float32) + tb2

    nb = n // NPB_C
    idx = idxr[...]
    ii = lax.broadcasted_iota(jnp.int32, (n, idx.shape[1]), 0)
    csel = jnp.zeros(ii.shape, jnp.bool_)
    for b in range(nb):
        csel = jnp.logical_or(csel, ii == (idx + b * NPB_C))
    M = v * csel.astype(jnp.float32)
    ib = lax.broadcasted_iota(jnp.int32, (nb, n), 0)
    ir = lax.broadcasted_iota(jnp.int32, (nb, n), 1)
    diff = ir - ib * NPB_C
    bsel = jnp.logical_and(diff >= 0, diff < NPB_C).astype(jnp.float32)
    sn = jnp.dot(bsel, M, preferred_element_type=jnp.float32)

    u = jnp.dot(sn, sW1, preferred_element_type=jnp.float32) + sb1
    mu1 = jnp.mean(u, axis=0, keepdims=True)
    var1 = jnp.mean((u - mu1) * (u - mu1), axis=0, keepdims=True)
    u = _leaky((u - mu1) * lax.rsqrt(var1 + 1e-5) * s1g + s1b)
    u = jnp.dot(u, sW2, preferred_element_type=jnp.float32) + sb2
    mu2 = jnp.mean(u, axis=0, keepdims=True)
    var2 = jnp.mean((u - mu2) * (u - mu2), axis=0, keepdims=True)
    u = _leaky((u - mu2) * lax.rsqrt(var2 + 1e-5) * s2g + s2b)
    out_ref[...] = jnp.dot(u, sW3, preferred_element_type=jnp.float32) + sb3


def _dense_call(x, p0, p1, cnt_parts, sn_idx, wlist):
    nb = x.shape[0] // NPB_C
    return pl.pallas_call(
        _dense_body,
        out_shape=jax.ShapeDtypeStruct((nb, 2), jnp.float32),
    )(x, p0, p1, cnt_parts, sn_idx, *wlist)


def kernel(x, edge_index, edge_attr, batch, supernode_indices, params):
    n, d = x.shape
    e = edge_attr.shape[0]
    info = plsc.get_sparse_core_info()
    nw = info.num_cores * info.num_subcores
    chunks = e // (nw * CHUNK)
    assert chunks * nw * CHUNK == e

    src3 = edge_index[0].reshape(nw, chunks, CHUNK)
    zeros_rows = np.zeros((CHUNK, d), np.float32)
    zeros_n = np.zeros((n,), np.float32)
    acc, cnt_parts = _sc_segment_sum(src3, edge_attr, zeros_rows, zeros_n, n)

    wlist = []
    for p in params["proc"]:
        wlist += [p["W1"], p["b1"].reshape(1, -1),
                  p["W2"], p["b2"].reshape(1, -1),
                  p["ln_g"].reshape(1, -1), p["ln_b"].reshape(1, -1)]
    t = params["time"]
    wlist += [t["W1"], t["b1"].reshape(1, -1), t["bn_g"].reshape(1, -1),
              t["bn_b"].reshape(1, -1), t["W2"], t["b2"].reshape(1, -1)]
    s = params["super"]
    wlist += [s["W1"], s["b1"].reshape(1, -1), s["bn1_g"].reshape(1, -1),
              s["bn1_b"].reshape(1, -1), s["W2"], s["b2"].reshape(1, -1),
              s["bn2_g"].reshape(1, -1), s["bn2_b"].reshape(1, -1),
              s["W3"], s["b3"].reshape(1, -1)]

    sn_idx = supernode_indices.reshape(1, -1).astype(jnp.int32)
    return _dense_call(x, acc[0], acc[1], cnt_parts, sn_idx, wlist)
